# baseline (speedup 1.0000x reference)
import functools

import jax
import jax.numpy as jnp
from jax import lax
from jax.experimental import pallas as pl
from jax.experimental.pallas import tpu as pltpu
from jax.experimental.pallas import tpu_sc as plsc

N_RAYS = 65536
N_BINS = 256
N_SAMPLES = 256
L = 16
NC, NSUB = 2, 16
NW = NC * NSUB
RAYS_PER_W = N_RAYS // NW
RB = 128
NBLK = RAYS_PER_W // RB
F32 = jnp.float32
I32 = jnp.int32


def _body(ox_h, oy_h, oz_h, dx_h, dy_h, dz_h, wt_h, out_h,
          ox_v, oy_v, oz_v, dx_v, dy_v, dz_v,
          near_v, hs_v, w_v, out_v, c0_v, c1_v, h0_v, h1_v, t0_v, t1_v):
    wid = lax.axis_index("c") * NSUB + lax.axis_index("s")
    ones_i = jnp.full((L,), 1, I32)
    zeros_i = jnp.full((L,), 0, I32)
    iota_i = lax.iota(I32, L)
    iota16 = iota_i * 16
    iota16_f = iota16.astype(F32)
    fifteen = jnp.full((L,), 15, I32)

    c_v = (c0_v, c1_v)
    hist_v = (h0_v, h1_v)
    tmp_v = (t0_v, t1_v)

    def block(blk, _):
        rbase = wid * RAYS_PER_W + blk * RB
        pltpu.sync_copy(wt_h.at[pl.ds(rbase * N_BINS, RB * N_BINS)], w_v)
        pltpu.sync_copy(ox_h.at[pl.ds(rbase, RB)], ox_v)
        pltpu.sync_copy(oy_h.at[pl.ds(rbase, RB)], oy_v)
        pltpu.sync_copy(oz_h.at[pl.ds(rbase, RB)], oz_v)
        pltpu.sync_copy(dx_h.at[pl.ds(rbase, RB)], dx_v)
        pltpu.sync_copy(dy_h.at[pl.ds(rbase, RB)], dy_v)
        pltpu.sync_copy(dz_h.at[pl.ds(rbase, RB)], dz_v)

        c0_v[pl.ds(0, L)] = jnp.full((L,), 0.0, F32)
        c1_v[pl.ds(0, L)] = jnp.full((L,), 0.0, F32)

        for g in range(RB // L):
            sl = pl.ds(g * L, L)
            lo = None
            hi = None
            for o_ref, d_ref in ((ox_v, dx_v), (oy_v, dy_v), (oz_v, dz_v)):
                o = o_ref[sl]
                d = d_ref[sl] + F32(1e-15)
                tmin = (F32(-2.0) - o) / d
                tmax = (F32(2.0) - o) / d
                a_lo = jnp.where(tmin < tmax, tmin, tmax)
                a_hi = jnp.where(tmin > tmax, tmin, tmax)
                lo = a_lo if lo is None else jnp.maximum(lo, a_lo)
                hi = a_hi if hi is None else jnp.minimum(hi, a_hi)
            bad = hi < lo
            nr = jnp.where(bad, F32(1e9), lo)
            fr = jnp.where(bad, F32(1e9), hi)
            nr = jnp.maximum(nr, F32(0.05))
            near_v[sl] = nr
            hs_v[sl] = (fr - nr) * F32(1.0 / 256.0)

        def ray(i, _):
            rr = (i * 2, i * 2 + 1)
            for p in range(2):
                for cc in range(17):
                    hist_v[p][pl.ds(cc * L, L)] = zeros_i

            offs = [None, None]
            inv_s = [None, None]
            for p in range(2):
                wbase = rr[p] * N_BINS
                tot = None
                for j in range(L):
                    vj = w_v[pl.ds(wbase + j * L, L)] + F32(1e-5)
                    tot = vj if tot is None else tot + vj
                cum_t = plsc.cumsum(tot)
                offs[p] = cum_t - tot
                tmp_v[p][pl.ds(0, L)] = cum_t
                s_spl = plsc.load_gather(tmp_v[p], [fifteen])
                inv_s[p] = F32(1.0) / s_spl

            for p in range(2):
                wbase = rr[p] * N_BINS
                acc = None
                for j in range(L):
                    vj = w_v[pl.ds(wbase + j * L, L)] + F32(1e-5)
                    acc = vj if acc is None else acc + vj
                    cs = (acc + offs[p]) * inv_s[p]
                    plsc.store_scatter(c_v[p], [iota16 + (j + 1)], cs)
                    m = cs * F32(256.0) - F32(0.5)
                    ti = m.astype(I32)
                    cl = ti + jnp.where(m > ti.astype(F32), 1, 0)
                    sidx = jnp.minimum(jnp.maximum(cl, 0), 256)
                    plsc.addupdate_scatter(hist_v[p], [sidx], ones_i)

            for p in range(2):
                rfull = jnp.full((L,), rr[p], I32)
                near_s = plsc.load_gather(near_v, [rfull])
                hs_s = plsc.load_gather(hs_v, [rfull])
                tot = None
                for j in range(L):
                    hj = plsc.load_gather(hist_v[p], [iota16 + j])
                    tot = hj if tot is None else tot + hj
                hoffs = plsc.cumsum(tot) - tot
                obase = rr[p] * N_SAMPLES
                acc = None
                for j in range(L):
                    hj = plsc.load_gather(hist_v[p], [iota16 + j])
                    acc = hj if acc is None else acc + hj
                    b = acc + hoffs
                    cgb = plsc.load_gather(c_v[p], [b])
                    ia = jnp.minimum(b + 1, 256)
                    cga = plsc.load_gather(c_v[p], [ia])
                    denom = cga - cgb
                    denom = jnp.where(denom < F32(1e-5), F32(1.0), denom)
                    u = (iota16_f + F32(j + 0.5)) * F32(1.0 / 256.0)
                    t = (u - cgb) / denom
                    y = b.astype(F32) + t * (ia - b).astype(F32)
                    plsc.store_scatter(
                        out_v, [iota16 + (obase + j)], near_s + hs_s * y)
            return 0

        lax.fori_loop(0, RB // 2, ray, 0)
        pltpu.sync_copy(out_v, out_h.at[pl.ds(rbase * N_SAMPLES, RB * N_SAMPLES)])
        return 0

    lax.fori_loop(0, NBLK, block, 0)


@jax.jit
def kernel(rays_o, rays_d, weights):
    mesh = plsc.VectorSubcoreMesh(core_axis_name="c", subcore_axis_name="s")
    k = functools.partial(
        pl.kernel,
        out_type=jax.ShapeDtypeStruct((N_RAYS * N_SAMPLES,), F32),
        mesh=mesh,
        compiler_params=pltpu.CompilerParams(needs_layout_passes=False),
        scratch_types=[
            pltpu.VMEM((RB,), F32),
            pltpu.VMEM((RB,), F32),
            pltpu.VMEM((RB,), F32),
            pltpu.VMEM((RB,), F32),
            pltpu.VMEM((RB,), F32),
            pltpu.VMEM((RB,), F32),
            pltpu.VMEM((RB,), F32),
            pltpu.VMEM((RB,), F32),
            pltpu.VMEM((RB * N_BINS,), F32),
            pltpu.VMEM((RB * N_SAMPLES,), F32),
            pltpu.VMEM((272,), F32),
            pltpu.VMEM((272,), F32),
            pltpu.VMEM((272,), I32),
            pltpu.VMEM((272,), I32),
            pltpu.VMEM((L,), F32),
            pltpu.VMEM((L,), F32),
        ],
    )(_body)
    wt = (weights.astype(F32)
          .reshape(N_RAYS, L, L).transpose(0, 2, 1).reshape(N_RAYS * N_BINS))
    out = k(
        rays_o[:, 0].astype(F32), rays_o[:, 1].astype(F32), rays_o[:, 2].astype(F32),
        rays_d[:, 0].astype(F32), rays_d[:, 1].astype(F32), rays_d[:, 2].astype(F32),
        wt,
    )
    return out.reshape(N_RAYS, N_SAMPLES)


# speedup vs baseline: 1.4883x; 1.4883x over previous
import functools

import jax
import jax.numpy as jnp
from jax import lax
from jax.experimental import pallas as pl
from jax.experimental.pallas import tpu as pltpu
from jax.experimental.pallas import tpu_sc as plsc

N_RAYS = 65536
N_BINS = 256
N_SAMPLES = 256
L = 16
NC, NSUB = 2, 16
NW = NC * NSUB
RAYS_PER_W = N_RAYS // NW
RB = 128
NBLK = RAYS_PER_W // RB
F32 = jnp.float32
I32 = jnp.int32


def _body(ox_h, oy_h, oz_h, dx_h, dy_h, dz_h, wt_h, out_h,
          ox_v, oy_v, oz_v, dx_v, dy_v, dz_v,
          near_v, hs_v, w_v, out_v, c0_v, c1_v, h0_v, h1_v, t0_v, t1_v):
    wid = lax.axis_index("c") * NSUB + lax.axis_index("s")
    ones_i = jnp.full((L,), 1, I32)
    zeros_i = jnp.full((L,), 0, I32)
    iota_i = lax.iota(I32, L)
    iota16 = iota_i * 16
    iota16_f = iota16.astype(F32)
    fifteen = jnp.full((L,), 15, I32)

    c_v = (c0_v, c1_v)
    hist_v = (h0_v, h1_v)
    tmp_v = (t0_v, t1_v)

    def block(blk, _):
        rbase = wid * RAYS_PER_W + blk * RB
        pltpu.sync_copy(wt_h.at[pl.ds(rbase * N_BINS, RB * N_BINS)], w_v)
        pltpu.sync_copy(ox_h.at[pl.ds(rbase, RB)], ox_v)
        pltpu.sync_copy(oy_h.at[pl.ds(rbase, RB)], oy_v)
        pltpu.sync_copy(oz_h.at[pl.ds(rbase, RB)], oz_v)
        pltpu.sync_copy(dx_h.at[pl.ds(rbase, RB)], dx_v)
        pltpu.sync_copy(dy_h.at[pl.ds(rbase, RB)], dy_v)
        pltpu.sync_copy(dz_h.at[pl.ds(rbase, RB)], dz_v)

        c0_v[pl.ds(0, L)] = jnp.full((L,), 0.0, F32)
        c1_v[pl.ds(0, L)] = jnp.full((L,), 0.0, F32)

        for g in range(RB // L):
            sl = pl.ds(g * L, L)
            lo = None
            hi = None
            for o_ref, d_ref in ((ox_v, dx_v), (oy_v, dy_v), (oz_v, dz_v)):
                o = o_ref[sl]
                d = d_ref[sl] + F32(1e-15)
                tmin = (F32(-2.0) - o) / d
                tmax = (F32(2.0) - o) / d
                a_lo = jnp.where(tmin < tmax, tmin, tmax)
                a_hi = jnp.where(tmin > tmax, tmin, tmax)
                lo = a_lo if lo is None else jnp.maximum(lo, a_lo)
                hi = a_hi if hi is None else jnp.minimum(hi, a_hi)
            bad = hi < lo
            nr = jnp.where(bad, F32(1e9), lo)
            fr = jnp.where(bad, F32(1e9), hi)
            nr = jnp.maximum(nr, F32(0.05))
            near_v[sl] = nr
            hs_v[sl] = (fr - nr) * F32(1.0 / 256.0)

        def ray(i, _):
            rr = (i * 2, i * 2 + 1)
            for p in range(2):
                for cc in range(17):
                    hist_v[p][pl.ds(cc * L, L)] = zeros_i

            for p in range(2):
                wbase = rr[p] * N_BINS
                accs = []
                acc = None
                for j in range(L):
                    vj = w_v[pl.ds(wbase + j * L, L)] + F32(1e-5)
                    acc = vj if acc is None else acc + vj
                    accs.append(acc)
                tot = accs[L - 1]
                cum_t = plsc.cumsum(tot)
                offs = cum_t - tot
                tmp_v[p][pl.ds(0, L)] = cum_t
                s_spl = plsc.load_gather(tmp_v[p], [fifteen])
                inv_s = F32(1.0) / s_spl
                for j in range(L):
                    cs = (accs[j] + offs) * inv_s
                    plsc.store_scatter(c_v[p], [iota16 + (j + 1)], cs)
                    m = cs * F32(256.0) - F32(0.5)
                    ti = m.astype(I32)
                    cl = ti + jnp.where(m > ti.astype(F32), 1, 0)
                    sidx = jnp.minimum(jnp.maximum(cl, 0), 256)
                    plsc.addupdate_scatter(hist_v[p], [sidx], ones_i)

            for p in range(2):
                rfull = jnp.full((L,), rr[p], I32)
                near_s = plsc.load_gather(near_v, [rfull])
                hs_s = plsc.load_gather(hs_v, [rfull])
                haccs = []
                acc = None
                for j in range(L):
                    hj = plsc.load_gather(hist_v[p], [iota16 + j])
                    acc = hj if acc is None else acc + hj
                    haccs.append(acc)
                tot = haccs[L - 1]
                hoffs = plsc.cumsum(tot) - tot
                obase = rr[p] * N_SAMPLES
                for j in range(L):
                    b = haccs[j] + hoffs
                    cgb = plsc.load_gather(c_v[p], [b])
                    ia = jnp.minimum(b + 1, 256)
                    cga = plsc.load_gather(c_v[p], [ia])
                    denom = cga - cgb
                    denom = jnp.where(denom < F32(1e-5), F32(1.0), denom)
                    u = (iota16_f + F32(j + 0.5)) * F32(1.0 / 256.0)
                    t = (u - cgb) / denom
                    y = b.astype(F32) + t * (ia - b).astype(F32)
                    plsc.store_scatter(
                        out_v, [iota16 + (obase + j)], near_s + hs_s * y)
            return 0

        lax.fori_loop(0, RB // 2, ray, 0)
        pltpu.sync_copy(out_v, out_h.at[pl.ds(rbase * N_SAMPLES, RB * N_SAMPLES)])
        return 0

    lax.fori_loop(0, NBLK, block, 0)


@jax.jit
def kernel(rays_o, rays_d, weights):
    mesh = plsc.VectorSubcoreMesh(core_axis_name="c", subcore_axis_name="s")
    k = functools.partial(
        pl.kernel,
        out_type=jax.ShapeDtypeStruct((N_RAYS * N_SAMPLES,), F32),
        mesh=mesh,
        compiler_params=pltpu.CompilerParams(needs_layout_passes=False),
        scratch_types=[
            pltpu.VMEM((RB,), F32),
            pltpu.VMEM((RB,), F32),
            pltpu.VMEM((RB,), F32),
            pltpu.VMEM((RB,), F32),
            pltpu.VMEM((RB,), F32),
            pltpu.VMEM((RB,), F32),
            pltpu.VMEM((RB,), F32),
            pltpu.VMEM((RB,), F32),
            pltpu.VMEM((RB * N_BINS,), F32),
            pltpu.VMEM((RB * N_SAMPLES,), F32),
            pltpu.VMEM((272,), F32),
            pltpu.VMEM((272,), F32),
            pltpu.VMEM((272,), I32),
            pltpu.VMEM((272,), I32),
            pltpu.VMEM((L,), F32),
            pltpu.VMEM((L,), F32),
        ],
    )(_body)
    wt = (weights.astype(F32)
          .reshape(N_RAYS, L, L).transpose(0, 2, 1).reshape(N_RAYS * N_BINS))
    out = k(
        rays_o[:, 0].astype(F32), rays_o[:, 1].astype(F32), rays_o[:, 2].astype(F32),
        rays_d[:, 0].astype(F32), rays_d[:, 1].astype(F32), rays_d[:, 2].astype(F32),
        wt,
    )
    return out.reshape(N_RAYS, N_SAMPLES)


# speedup vs baseline: 1.9598x; 1.3168x over previous
import functools

import jax
import jax.numpy as jnp
from jax import lax
from jax.experimental import pallas as pl
from jax.experimental.pallas import tpu as pltpu
from jax.experimental.pallas import tpu_sc as plsc

N_RAYS = 65536
N_BINS = 256
N_SAMPLES = 256
L = 16
NC, NSUB = 2, 16
NW = NC * NSUB
RAYS_PER_W = N_RAYS // NW
RB = 128
NBLK = RAYS_PER_W // RB
F32 = jnp.float32
I32 = jnp.int32


def _body(ro_h, rd_h, w_h, out_h,
          ro_v, rd_v,
          near_v, hs_v, w_v, out_v, c0_v, c1_v, h0_v, h1_v, t0_v, t1_v):
    wid = lax.axis_index("c") * NSUB + lax.axis_index("s")
    ones_i = jnp.full((L,), 1, I32)
    zeros_i = jnp.full((L,), 0, I32)
    iota_i = lax.iota(I32, L)
    iota16 = iota_i * 16
    iota16_f = iota16.astype(F32)
    iota3 = iota_i * 3
    fifteen = jnp.full((L,), 15, I32)

    c_v = (c0_v, c1_v)
    hist_v = (h0_v, h1_v)
    tmp_v = (t0_v, t1_v)

    def block(blk, _):
        rbase = wid * RAYS_PER_W + blk * RB
        pltpu.sync_copy(w_h.at[pl.ds(rbase * N_BINS, RB * N_BINS)], w_v)
        pltpu.sync_copy(ro_h.at[pl.ds(rbase * 3, RB * 3)], ro_v)
        pltpu.sync_copy(rd_h.at[pl.ds(rbase * 3, RB * 3)], rd_v)

        c0_v[pl.ds(0, L)] = jnp.full((L,), 0.0, F32)
        c1_v[pl.ds(0, L)] = jnp.full((L,), 0.0, F32)

        for g in range(RB // L):
            sl = pl.ds(g * L, L)
            lo = None
            hi = None
            for ax in range(3):
                idx3 = iota3 + (g * (3 * L) + ax)
                o = plsc.load_gather(ro_v, [idx3])
                d = plsc.load_gather(rd_v, [idx3]) + F32(1e-15)
                tmin = (F32(-2.0) - o) / d
                tmax = (F32(2.0) - o) / d
                a_lo = jnp.where(tmin < tmax, tmin, tmax)
                a_hi = jnp.where(tmin > tmax, tmin, tmax)
                lo = a_lo if lo is None else jnp.maximum(lo, a_lo)
                hi = a_hi if hi is None else jnp.minimum(hi, a_hi)
            bad = hi < lo
            nr = jnp.where(bad, F32(1e9), lo)
            fr = jnp.where(bad, F32(1e9), hi)
            nr = jnp.maximum(nr, F32(0.05))
            near_v[sl] = nr
            hs_v[sl] = (fr - nr) * F32(1.0 / 256.0)

        def ray(i, _):
            rr = (i * 2, i * 2 + 1)
            for p in range(2):
                for cc in range(17):
                    hist_v[p][pl.ds(cc * L, L)] = zeros_i

            for p in range(2):
                wbase = rr[p] * N_BINS
                widx = iota16 + wbase
                accs = []
                acc = None
                for j in range(L):
                    vj = plsc.load_gather(w_v, [widx + j]) + F32(1e-5)
                    acc = vj if acc is None else acc + vj
                    accs.append(acc)
                tot = accs[L - 1]
                cum_t = plsc.cumsum(tot)
                offs = cum_t - tot
                tmp_v[p][pl.ds(0, L)] = cum_t
                s_spl = plsc.load_gather(tmp_v[p], [fifteen])
                inv_s = F32(1.0) / s_spl
                for j in range(L):
                    cs = (accs[j] + offs) * inv_s
                    plsc.store_scatter(c_v[p], [iota16 + (j + 1)], cs)
                    m = cs * F32(256.0) - F32(0.5)
                    ti = m.astype(I32)
                    cl = ti + jnp.where(m > ti.astype(F32), 1, 0)
                    sidx = jnp.minimum(jnp.maximum(cl, 0), 256)
                    plsc.addupdate_scatter(hist_v[p], [sidx], ones_i)

            for p in range(2):
                rfull = jnp.full((L,), rr[p], I32)
                near_s = plsc.load_gather(near_v, [rfull])
                hs_s = plsc.load_gather(hs_v, [rfull])
                haccs = []
                acc = None
                for j in range(L):
                    hj = plsc.load_gather(hist_v[p], [iota16 + j])
                    acc = hj if acc is None else acc + hj
                    haccs.append(acc)
                tot = haccs[L - 1]
                hoffs = plsc.cumsum(tot) - tot
                obase = rr[p] * N_SAMPLES
                for j in range(L):
                    b = haccs[j] + hoffs
                    cgb = plsc.load_gather(c_v[p], [b])
                    ia = jnp.minimum(b + 1, 256)
                    cga = plsc.load_gather(c_v[p], [ia])
                    denom = cga - cgb
                    denom = jnp.where(denom < F32(1e-5), F32(1.0), denom)
                    u = (iota16_f + F32(j + 0.5)) * F32(1.0 / 256.0)
                    t = (u - cgb) / denom
                    y = b.astype(F32) + t * (ia - b).astype(F32)
                    plsc.store_scatter(
                        out_v, [iota16 + (obase + j)], near_s + hs_s * y)
            return 0

        lax.fori_loop(0, RB // 2, ray, 0)
        pltpu.sync_copy(out_v, out_h.at[pl.ds(rbase * N_SAMPLES, RB * N_SAMPLES)])
        return 0

    lax.fori_loop(0, NBLK, block, 0)


@jax.jit
def kernel(rays_o, rays_d, weights):
    mesh = plsc.VectorSubcoreMesh(core_axis_name="c", subcore_axis_name="s")
    k = functools.partial(
        pl.kernel,
        out_type=jax.ShapeDtypeStruct((N_RAYS * N_SAMPLES,), F32),
        mesh=mesh,
        compiler_params=pltpu.CompilerParams(needs_layout_passes=False),
        scratch_types=[
            pltpu.VMEM((RB * 3,), F32),
            pltpu.VMEM((RB * 3,), F32),
            pltpu.VMEM((RB,), F32),
            pltpu.VMEM((RB,), F32),
            pltpu.VMEM((RB * N_BINS,), F32),
            pltpu.VMEM((RB * N_SAMPLES,), F32),
            pltpu.VMEM((272,), F32),
            pltpu.VMEM((272,), F32),
            pltpu.VMEM((272,), I32),
            pltpu.VMEM((272,), I32),
            pltpu.VMEM((L,), F32),
            pltpu.VMEM((L,), F32),
        ],
    )(_body)
    out = k(
        rays_o.astype(F32).reshape(N_RAYS * 3),
        rays_d.astype(F32).reshape(N_RAYS * 3),
        weights.astype(F32).reshape(N_RAYS * N_BINS),
    )
    return out.reshape(N_RAYS, N_SAMPLES)
